# baseline (speedup 1.0000x reference)
import functools

import jax
import jax.numpy as jnp
from jax import lax
from jax.experimental import pallas as pl
from jax.experimental.pallas import tpu as pltpu
from jax.experimental.pallas import tpu_sc as plsc

NC = 2
NS = 16
NW = NC * NS
L = 16

BLK = 64
N_ATOMS = 100000
NB_FULL = N_ATOMS // BLK
TAIL = N_ATOMS - NB_FULL * BLK
TAIL_OFF = NB_FULL * BLK
BPW = -(-NB_FULL // NW)
STAGE = BPW * BLK
NBUF = 6
AHEAD = 3

T_ROWS = 384
ROWS_PER_TILE = T_ROWS // NS
ZPT = ROWS_PER_TILE // 3
BUILD_TILES = -(-303 // ROWS_PER_TILE)

D_BASE = 224
D_TAG = 32
D = D_BASE + D_TAG


def kernel(z, tag, base_table, tag_table):
    zi = z.astype(jnp.int32)
    ti = tag.astype(jnp.int32)
    base_pad = jnp.pad(
        base_table,
        ((0, BUILD_TILES * ZPT - base_table.shape[0]), (0, 256 - D_BASE)))
    tag_pad = jnp.pad(
        tag_table, ((0, 8 - tag_table.shape[0]), (0, 128 - D_TAG)))

    mesh = plsc.VectorSubcoreMesh(
        core_axis_name="c", subcore_axis_name="s",
        num_cores=NC, num_subcores=NS)

    @functools.partial(
        pl.kernel,
        out_type=(jax.ShapeDtypeStruct((N_ATOMS, D), jnp.float32),
                  jax.ShapeDtypeStruct((T_ROWS, D), jnp.float32)),
        mesh=mesh,
        compiler_params=pltpu.CompilerParams(use_tc_tiling_on_sc=True),
        scratch_types=[
            pltpu.VMEM((STAGE,), jnp.int32),
            pltpu.VMEM((STAGE,), jnp.int32),
            pltpu.VMEM((STAGE,), jnp.int32),
            pltpu.VMEM((ZPT, 256), jnp.float32),
            pltpu.VMEM((8, 128), jnp.float32),
            pltpu.VMEM((ROWS_PER_TILE, D), jnp.float32),
            [pltpu.VMEM((BLK, D), jnp.float32) for _ in range(NBUF)],
            [pltpu.SemaphoreType.DMA for _ in range(NBUF)],
            [pltpu.SemaphoreType.DMA for _ in range(NBUF)],
            pltpu.SemaphoreType.DMA,
        ],
    )
    def sc_kernel(z_hbm, t_hbm, base_hbm, tagtab_hbm, out_hbm, ctable_hbm,
                  zv, tv, civ, basev, tagv, bld, bufs, gsems, wsems, tsem):
        cid = lax.axis_index("c")
        sid = lax.axis_index("s")
        wid = sid * NC + cid
        lo = (wid * NB_FULL) >> 5
        base_atom = pl.multiple_of(lo * BLK, BLK)

        pltpu.sync_copy(z_hbm.at[pl.ds(base_atom, STAGE)], zv)
        pltpu.sync_copy(t_hbm.at[pl.ds(base_atom, STAGE)], tv)
        for k in range(STAGE // L):
            s = pl.ds(k * L, L)
            civ[s] = zv[s] * 3 + tv[s]

        @pl.when(sid < BUILD_TILES)
        def _build():
            zrow0 = pl.multiple_of(sid * ZPT, ZPT)
            pltpu.sync_copy(base_hbm.at[pl.ds(zrow0, ZPT), :], basev)
            pltpu.sync_copy(tagtab_hbm, tagv)
            for r in range(ZPT):
                for rep in range(3):
                    row = 3 * r + rep
                    for c in range(D_BASE // L):
                        bld[row, pl.ds(c * L, L)] = basev[r, pl.ds(c * L, L)]
                    for c in range(D_TAG // L):
                        bld[row, pl.ds(D_BASE + c * L, L)] = (
                            tagv[rep, pl.ds(c * L, L)])
            crow0 = pl.multiple_of(sid * ROWS_PER_TILE, ROWS_PER_TILE)
            pltpu.sync_copy(bld, ctable_hbm.at[pl.ds(crow0, ROWS_PER_TILE), :])

        plsc.subcore_barrier()

        def start_gather(t):
            b = t % NBUF
            return pltpu.async_copy(
                ctable_hbm.at[civ.at[pl.ds(t * BLK, BLK)]], bufs[b], gsems[b])

        def start_write(t):
            b = t % NBUF
            row0 = pl.multiple_of(base_atom + t * BLK, BLK)
            return pltpu.async_copy(
                bufs[b], out_hbm.at[pl.ds(row0, BLK), :], wsems[b])

        gathers = {t: start_gather(t) for t in range(min(AHEAD, BPW))}
        writes = {}
        for t in range(BPW):
            gathers.pop(t).wait()
            tw = t - (NBUF - AHEAD)
            if tw >= 0:
                writes.pop(tw).wait()
            if t + AHEAD < BPW:
                gathers[t + AHEAD] = start_gather(t + AHEAD)
            writes[t] = start_write(t)
        for tw in sorted(writes):
            writes.pop(tw).wait()

        @pl.when(wid == 0)
        def _tail():
            pltpu.sync_copy(z_hbm.at[pl.ds(TAIL_OFF, TAIL)], zv.at[pl.ds(0, TAIL)])
            pltpu.sync_copy(t_hbm.at[pl.ds(TAIL_OFF, TAIL)], tv.at[pl.ds(0, TAIL)])
            for k in range(TAIL // L):
                s = pl.ds(k * L, L)
                civ[s] = zv[s] * 3 + tv[s]
            pltpu.async_copy(
                ctable_hbm.at[civ.at[pl.ds(0, TAIL)]],
                bufs[0].at[pl.ds(0, TAIL), :], tsem).wait()
            pltpu.sync_copy(
                bufs[0].at[pl.ds(0, TAIL), :],
                out_hbm.at[pl.ds(TAIL_OFF, TAIL), :])

    out, _ = sc_kernel(zi, ti, base_pad, tag_pad)
    return out


# speedup vs baseline: 1.0014x; 1.0014x over previous
import functools

import jax
import jax.numpy as jnp
from jax import lax
from jax.experimental import pallas as pl
from jax.experimental.pallas import tpu as pltpu
from jax.experimental.pallas import tpu_sc as plsc

NC = 2
NS = 16
NW = NC * NS
L = 16

BLK = 64
N_ATOMS = 100000
NB_FULL = N_ATOMS // BLK
TAIL = N_ATOMS - NB_FULL * BLK
TAIL_OFF = NB_FULL * BLK
BPW = -(-NB_FULL // NW)
STAGE = BPW * BLK
NBUF = 6
AHEAD = 4

T_ROWS = 384
ROWS_PER_TILE = T_ROWS // NS
ZPT = ROWS_PER_TILE // 3
BUILD_TILES = -(-303 // ROWS_PER_TILE)

D_BASE = 224
D_TAG = 32
D = D_BASE + D_TAG


def kernel(z, tag, base_table, tag_table):
    zi = z.astype(jnp.int32)
    ti = tag.astype(jnp.int32)
    base_pad = jnp.pad(
        base_table,
        ((0, BUILD_TILES * ZPT - base_table.shape[0]), (0, 256 - D_BASE)))
    tag_pad = jnp.pad(
        tag_table, ((0, 8 - tag_table.shape[0]), (0, 128 - D_TAG)))

    mesh = plsc.VectorSubcoreMesh(
        core_axis_name="c", subcore_axis_name="s",
        num_cores=NC, num_subcores=NS)

    @functools.partial(
        pl.kernel,
        out_type=(jax.ShapeDtypeStruct((N_ATOMS, D), jnp.float32),
                  jax.ShapeDtypeStruct((T_ROWS, D), jnp.float32)),
        mesh=mesh,
        compiler_params=pltpu.CompilerParams(use_tc_tiling_on_sc=True),
        scratch_types=[
            pltpu.VMEM((STAGE,), jnp.int32),
            pltpu.VMEM((STAGE,), jnp.int32),
            pltpu.VMEM((STAGE,), jnp.int32),
            pltpu.VMEM((ZPT, 256), jnp.float32),
            pltpu.VMEM((8, 128), jnp.float32),
            pltpu.VMEM((ROWS_PER_TILE, D), jnp.float32),
            [pltpu.VMEM((BLK, D), jnp.float32) for _ in range(NBUF)],
            [pltpu.SemaphoreType.DMA for _ in range(NBUF)],
            [pltpu.SemaphoreType.DMA for _ in range(NBUF)],
            pltpu.SemaphoreType.DMA,
        ],
    )
    def sc_kernel(z_hbm, t_hbm, base_hbm, tagtab_hbm, out_hbm, ctable_hbm,
                  zv, tv, civ, basev, tagv, bld, bufs, gsems, wsems, tsem):
        cid = lax.axis_index("c")
        sid = lax.axis_index("s")
        wid = sid * NC + cid
        lo = (wid * NB_FULL) >> 5
        base_atom = pl.multiple_of(lo * BLK, BLK)

        pltpu.sync_copy(z_hbm.at[pl.ds(base_atom, STAGE)], zv)
        pltpu.sync_copy(t_hbm.at[pl.ds(base_atom, STAGE)], tv)
        for k in range(STAGE // L):
            s = pl.ds(k * L, L)
            civ[s] = zv[s] * 3 + tv[s]

        @pl.when(sid < BUILD_TILES)
        def _build():
            zrow0 = pl.multiple_of(sid * ZPT, ZPT)
            pltpu.sync_copy(base_hbm.at[pl.ds(zrow0, ZPT), :], basev)
            pltpu.sync_copy(tagtab_hbm, tagv)
            for r in range(ZPT):
                for rep in range(3):
                    row = 3 * r + rep
                    for c in range(D_BASE // L):
                        bld[row, pl.ds(c * L, L)] = basev[r, pl.ds(c * L, L)]
                    for c in range(D_TAG // L):
                        bld[row, pl.ds(D_BASE + c * L, L)] = (
                            tagv[rep, pl.ds(c * L, L)])
            crow0 = pl.multiple_of(sid * ROWS_PER_TILE, ROWS_PER_TILE)
            pltpu.sync_copy(bld, ctable_hbm.at[pl.ds(crow0, ROWS_PER_TILE), :])

        plsc.subcore_barrier()

        def start_gather(t):
            b = t % NBUF
            return pltpu.async_copy(
                ctable_hbm.at[civ.at[pl.ds(t * BLK, BLK)]], bufs[b], gsems[b])

        def start_write(t):
            b = t % NBUF
            row0 = pl.multiple_of(base_atom + t * BLK, BLK)
            return pltpu.async_copy(
                bufs[b], out_hbm.at[pl.ds(row0, BLK), :], wsems[b])

        gathers = {t: start_gather(t) for t in range(min(AHEAD, BPW))}
        writes = {}
        for t in range(BPW):
            gathers.pop(t).wait()
            tw = t - (NBUF - AHEAD)
            if tw >= 0:
                writes.pop(tw).wait()
            if t + AHEAD < BPW:
                gathers[t + AHEAD] = start_gather(t + AHEAD)
            writes[t] = start_write(t)
        for tw in sorted(writes):
            writes.pop(tw).wait()

        @pl.when(wid == 0)
        def _tail():
            pltpu.sync_copy(z_hbm.at[pl.ds(TAIL_OFF, TAIL)], zv.at[pl.ds(0, TAIL)])
            pltpu.sync_copy(t_hbm.at[pl.ds(TAIL_OFF, TAIL)], tv.at[pl.ds(0, TAIL)])
            for k in range(TAIL // L):
                s = pl.ds(k * L, L)
                civ[s] = zv[s] * 3 + tv[s]
            pltpu.async_copy(
                ctable_hbm.at[civ.at[pl.ds(0, TAIL)]],
                bufs[0].at[pl.ds(0, TAIL), :], tsem).wait()
            pltpu.sync_copy(
                bufs[0].at[pl.ds(0, TAIL), :],
                out_hbm.at[pl.ds(TAIL_OFF, TAIL), :])

    out, _ = sc_kernel(zi, ti, base_pad, tag_pad)
    return out


# speedup vs baseline: 1.2945x; 1.2927x over previous
import functools

import jax
import jax.numpy as jnp
from jax import lax
from jax.experimental import pallas as pl
from jax.experimental.pallas import tpu as pltpu
from jax.experimental.pallas import tpu_sc as plsc

NC = 2
NS = 16
NW = NC * NS
L = 16

BLK = 64
N_ATOMS = 100000
NB_FULL = N_ATOMS // BLK
TAIL = N_ATOMS - NB_FULL * BLK
TAIL_OFF = NB_FULL * BLK
BPW = -(-NB_FULL // NW)
STAGE = BPW * BLK
NBUF = 6
AHEAD = 4

T_HALF = 384
T_ROWS = NC * T_HALF
ROWS_PER_TILE = T_HALF // NS
ZPT = ROWS_PER_TILE // 3
BUILD_TILES = -(-303 // ROWS_PER_TILE)

D_BASE = 224
D_TAG = 32
D = D_BASE + D_TAG

_MESH = plsc.VectorSubcoreMesh(
    core_axis_name="c", subcore_axis_name="s",
    num_cores=NC, num_subcores=NS)
_PARAMS = pltpu.CompilerParams(use_tc_tiling_on_sc=True)


def _build_table(base_pad, tag_pad):
    @functools.partial(
        pl.kernel,
        out_type=jax.ShapeDtypeStruct((T_ROWS, D), jnp.float32),
        mesh=_MESH,
        compiler_params=_PARAMS,
        scratch_types=[
            pltpu.VMEM((ZPT, 256), jnp.float32),
            pltpu.VMEM((8, 128), jnp.float32),
            pltpu.VMEM((ROWS_PER_TILE, D), jnp.float32),
        ],
    )
    def build(base_hbm, tagtab_hbm, ctable_hbm, basev, tagv, bld):
        cid = lax.axis_index("c")
        sid = lax.axis_index("s")

        @pl.when(sid < BUILD_TILES)
        def _build():
            zrow0 = pl.multiple_of(sid * ZPT, ZPT)
            pltpu.sync_copy(base_hbm.at[pl.ds(zrow0, ZPT), :], basev)
            pltpu.sync_copy(tagtab_hbm, tagv)
            for r in range(ZPT):
                for rep in range(3):
                    row = 3 * r + rep
                    for c in range(D_BASE // L):
                        bld[row, pl.ds(c * L, L)] = basev[r, pl.ds(c * L, L)]
                    for c in range(D_TAG // L):
                        bld[row, pl.ds(D_BASE + c * L, L)] = (
                            tagv[rep, pl.ds(c * L, L)])
            crow0 = pl.multiple_of(
                cid * T_HALF + sid * ROWS_PER_TILE, ROWS_PER_TILE)
            pltpu.sync_copy(bld, ctable_hbm.at[pl.ds(crow0, ROWS_PER_TILE), :])

    return build(base_pad, tag_pad)


def _gather(zi, ti, ctable):
    @functools.partial(
        pl.kernel,
        out_type=jax.ShapeDtypeStruct((N_ATOMS, D), jnp.float32),
        mesh=_MESH,
        compiler_params=_PARAMS,
        scratch_types=[
            pltpu.VMEM((STAGE,), jnp.int32),
            pltpu.VMEM((STAGE,), jnp.int32),
            pltpu.VMEM((STAGE,), jnp.int32),
            [pltpu.VMEM((BLK, D), jnp.float32) for _ in range(NBUF)],
            [pltpu.SemaphoreType.DMA for _ in range(NBUF)],
            [pltpu.SemaphoreType.DMA for _ in range(NBUF)],
            pltpu.SemaphoreType.DMA,
        ],
    )
    def gather(z_hbm, t_hbm, ctable_hbm, out_hbm, zv, tv, civ,
               bufs, gsems, wsems, tsem):
        cid = lax.axis_index("c")
        sid = lax.axis_index("s")
        wid = sid * NC + cid
        lo = (wid * NB_FULL) >> 5
        hi = ((wid + 1) * NB_FULL) >> 5
        owns_last = lo + BPW <= hi
        base_atom = pl.multiple_of(lo * BLK, BLK)
        coff = cid * T_HALF

        pltpu.sync_copy(z_hbm.at[pl.ds(base_atom, STAGE)], zv)
        pltpu.sync_copy(t_hbm.at[pl.ds(base_atom, STAGE)], tv)
        for k in range(STAGE // L):
            s = pl.ds(k * L, L)
            civ[s] = zv[s] * 3 + tv[s] + coff

        def gather_cp(t):
            b = t % NBUF
            return pltpu.make_async_copy(
                ctable_hbm.at[civ.at[pl.ds(t * BLK, BLK)]], bufs[b], gsems[b])

        def write_cp(t):
            b = t % NBUF
            row0 = pl.multiple_of(base_atom + t * BLK, BLK)
            return pltpu.make_async_copy(
                bufs[b], out_hbm.at[pl.ds(row0, BLK), :], wsems[b])

        def guarded(t, fn):
            if t == BPW - 1:
                pl.when(owns_last)(fn)
            else:
                fn()

        gathers = {}
        writes = {}
        for t in range(AHEAD):
            gathers[t] = gather_cp(t)
            guarded(t, gathers[t].start)
        for t in range(BPW):
            g = gathers.pop(t)
            guarded(t, g.wait)
            tw = t - (NBUF - AHEAD)
            if tw >= 0:
                w = writes.pop(tw)
                guarded(tw, w.wait)
            if t + AHEAD < BPW:
                cp = gather_cp(t + AHEAD)
                gathers[t + AHEAD] = cp
                guarded(t + AHEAD, cp.start)
            writes[t] = write_cp(t)
            guarded(t, writes[t].start)
        for tw in sorted(writes):
            w = writes.pop(tw)
            guarded(tw, w.wait)

        @pl.when(wid == 0)
        def _tail():
            pltpu.sync_copy(z_hbm.at[pl.ds(TAIL_OFF, TAIL)], zv.at[pl.ds(0, TAIL)])
            pltpu.sync_copy(t_hbm.at[pl.ds(TAIL_OFF, TAIL)], tv.at[pl.ds(0, TAIL)])
            for k in range(TAIL // L):
                s = pl.ds(k * L, L)
                civ[s] = zv[s] * 3 + tv[s] + coff
            pltpu.async_copy(
                ctable_hbm.at[civ.at[pl.ds(0, TAIL)]],
                bufs[0].at[pl.ds(0, TAIL), :], tsem).wait()
            pltpu.sync_copy(
                bufs[0].at[pl.ds(0, TAIL), :],
                out_hbm.at[pl.ds(TAIL_OFF, TAIL), :])

    return gather(zi, ti, ctable)


def kernel(z, tag, base_table, tag_table):
    zi = z.astype(jnp.int32)
    ti = tag.astype(jnp.int32)
    base_pad = jnp.pad(
        base_table,
        ((0, BUILD_TILES * ZPT - base_table.shape[0]), (0, 256 - D_BASE)))
    tag_pad = jnp.pad(
        tag_table, ((0, 8 - tag_table.shape[0]), (0, 128 - D_TAG)))
    ctable = _build_table(base_pad, tag_pad)
    return _gather(zi, ti, ctable)
